# baseline (speedup 1.0000x reference)
import functools
import jax
import jax.numpy as jnp
from jax import lax
from jax.experimental import pallas as pl
from jax.experimental.pallas import tpu as pltpu
from jax.experimental.pallas import tpu_sc as plsc

_T = 2048
_H = 1024
_HW = 512
_FF = 4096
_E = 8
_P = 4096
_B = 256
_NPAD = 5888
_NB = 23
_NTILE = 32
_RPT = _NPAD // _NTILE
_TPT = _T // _NTILE

_SC_PARAMS = pltpu.CompilerParams(needs_layout_passes=False)

_mesh = lambda: plsc.VectorSubcoreMesh(
    core_axis_name="c", subcore_axis_name="s", num_cores=2, num_subcores=16)


def _wid():
    return lax.axis_index("s") * 2 + lax.axis_index("c")


def _routing_body(ei_hbm, ew_hbm, srctok_hbm, slotw_hbm, pos0_hbm, pos1_hbm,
                  be_hbm, ei_v, ew_v, srctok_v, slotw_v, pos0_v, pos1_v, be_v,
                  cnt_v, poscnt_v):
    @pl.when(_wid() == 0)
    def _():
        lane = lax.iota(jnp.int32, 16)
        zeros16 = jnp.zeros((16,), jnp.int32)
        pltpu.sync_copy(ei_hbm, ei_v)
        pltpu.sync_copy(ew_hbm, ew_v)
        cnt_v[...] = zeros16

        def hist_step(c, _):
            chunk = ei_v[pl.ds(c * 16, 16)]
            upd = zeros16
            for e in range(_E):
                tot = plsc.all_reduce_population_count(chunk == e)
                upd = upd + jnp.where(lane == e, tot, 0)
            cnt_v[...] = cnt_v[...] + upd
            return _
        lax.fori_loop(0, _P // 16, hist_step, None)

        cnt = cnt_v[...]
        pcnt = ((cnt + (_B - 1)) >> 8) << 8
        incl = plsc.cumsum(pcnt)
        excl = incl - pcnt
        poscnt_v[...] = excl

        for cidx in range(2):
            bevec = zeros16
            for jj in range(16):
                j = cidx * 16 + jj
                cmp = (incl <= j * _B) & (lane < _E)
                cntj = plsc.all_reduce_population_count(cmp)
                bevec = bevec + jnp.where(lane == jj,
                                          jnp.minimum(cntj, _E - 1), 0)
            be_v[pl.ds(cidx * 16, 16)] = bevec

        def zero_step(c, _):
            srctok_v[pl.ds(c * 16, 16)] = zeros16
            slotw_v[pl.ds(c * 16, 16)] = jnp.zeros((16,), jnp.float32)
            return _
        lax.fori_loop(0, _NPAD // 16, zero_step, None)

        def pos_step(c, _):
            chunk = ei_v[pl.ds(c * 16, 16)]
            ewc = ew_v[pl.ds(c * 16, 16)]
            base = plsc.load_gather(poscnt_v, [chunk])
            r = zeros16
            upd = zeros16
            for e in range(_E):
                m = chunk == e
                pref = plsc.cumsum(m.astype(jnp.int32))
                r = r + jnp.where(m, pref - 1, 0)
                tot = plsc.all_reduce_population_count(m)
                upd = upd + jnp.where(lane == e, tot, 0)
            pos = base + r
            poscnt_v[...] = poscnt_v[...] + upd
            pairidx = c * 16 + lane
            tvec = pairidx >> 1
            kpar = pairidx & 1
            plsc.store_scatter(srctok_v, [pos], tvec)
            plsc.store_scatter(slotw_v, [pos], ewc)
            plsc.store_scatter(pos0_v, [tvec], pos, mask=kpar == 0)
            plsc.store_scatter(pos1_v, [tvec], pos, mask=kpar == 1)
            return _
        lax.fori_loop(0, _P // 16, pos_step, None)

        pltpu.sync_copy(srctok_v, srctok_hbm)
        pltpu.sync_copy(slotw_v, slotw_hbm)
        pltpu.sync_copy(pos0_v, pos0_hbm)
        pltpu.sync_copy(pos1_v, pos1_hbm)
        pltpu.sync_copy(be_v, be_hbm)


def _routing(ei, ew):
    return pl.kernel(
        _routing_body,
        out_type=(
            jax.ShapeDtypeStruct((_NPAD,), jnp.int32),
            jax.ShapeDtypeStruct((_NPAD,), jnp.float32),
            jax.ShapeDtypeStruct((_T,), jnp.int32),
            jax.ShapeDtypeStruct((_T,), jnp.int32),
            jax.ShapeDtypeStruct((32,), jnp.int32),
        ),
        mesh=_mesh(),
        scratch_types=[
            pltpu.VMEM((_P,), jnp.int32),
            pltpu.VMEM((_P,), jnp.float32),
            pltpu.VMEM((_NPAD,), jnp.int32),
            pltpu.VMEM((_NPAD,), jnp.float32),
            pltpu.VMEM((_T,), jnp.int32),
            pltpu.VMEM((_T,), jnp.int32),
            pltpu.VMEM((32,), jnp.int32),
            pltpu.VMEM((16,), jnp.int32),
            pltpu.VMEM((16,), jnp.int32),
        ],
        compiler_params=_SC_PARAMS,
    )(ei, ew)


def _gather_body(xi_hbm, srctok_hbm, xg_hbm, idx_a, idx_b, rows_a, rows_b,
                 sem_a, sem_b):
    base = _wid() * _RPT
    pltpu.sync_copy(srctok_hbm.at[pl.ds(base, 96)], idx_a)
    pltpu.sync_copy(srctok_hbm.at[pl.ds(base + 96, 88)], idx_b)
    ca = pltpu.async_copy(xi_hbm.at[idx_a], rows_a, sem_a)
    cb = pltpu.async_copy(xi_hbm.at[idx_b], rows_b, sem_b)
    ca.wait()
    pltpu.sync_copy(rows_a, xg_hbm.at[pl.ds(base, 96)])
    cb.wait()
    pltpu.sync_copy(rows_b, xg_hbm.at[pl.ds(base + 96, 88)])


def _gather(xi, srctok):
    return pl.kernel(
        _gather_body,
        out_type=jax.ShapeDtypeStruct((_NPAD, _HW), jnp.int32),
        mesh=_mesh(),
        scratch_types=[
            pltpu.VMEM((96,), jnp.int32),
            pltpu.VMEM((88,), jnp.int32),
            pltpu.VMEM((96, _HW), jnp.int32),
            pltpu.VMEM((88, _HW), jnp.int32),
            pltpu.SemaphoreType.DMA,
            pltpu.SemaphoreType.DMA,
        ],
        compiler_params=_SC_PARAMS,
    )(xi, srctok)


def _gemm_body(be_ref, xg_ref, w1_ref, w2_ref, sw_ref, o_ref, acc_ref):
    k = pl.program_id(0)
    i = pl.program_id(1)
    rows = pl.ds(i * _B, _B)

    @pl.when(k == 0)
    def _init():
        acc_ref[rows, :] = jnp.zeros((_B, _H), jnp.float32)

    h1 = jnp.dot(xg_ref[...], w1_ref[0], preferred_element_type=jnp.float32)
    h1 = jax.nn.gelu(h1)
    part = jnp.dot(h1.astype(jnp.bfloat16), w2_ref[0],
                   preferred_element_type=jnp.float32)
    acc_ref[rows, :] += part

    @pl.when(k == _FF // _H - 1)
    def _flush():
        sw = sw_ref[0, 0, :]
        o_ref[...] = acc_ref[rows, :] * sw[:, None]


def _gemm(xg, w1b, w2b, sw3, be):
    return pl.pallas_call(
        _gemm_body,
        grid_spec=pltpu.PrefetchScalarGridSpec(
            num_scalar_prefetch=1,
            grid=(_FF // _H, _NB),
            in_specs=[
                pl.BlockSpec((_B, _H), lambda k, i, be: (i, 0)),
                pl.BlockSpec((1, _H, _H), lambda k, i, be: (be[i], 0, k)),
                pl.BlockSpec((1, _H, _H), lambda k, i, be: (be[i], k, 0)),
                pl.BlockSpec((1, 1, _B), lambda k, i, be: (i, 0, 0)),
            ],
            out_specs=pl.BlockSpec((_B, _H), lambda k, i, be: (i, 0)),
            scratch_shapes=[pltpu.VMEM((_NPAD, _H), jnp.float32)],
        ),
        out_shape=jax.ShapeDtypeStruct((_NPAD, _H), jnp.float32),
        compiler_params=pltpu.CompilerParams(
            dimension_semantics=("arbitrary", "arbitrary"),
        ),
    )(be, xg, w1b, w2b, sw3)


def _combine_body(rows_hbm, pos0_hbm, pos1_hbm, out_hbm, idx_a, idx_b,
                  rows_a, rows_b, out_v, sem_a, sem_b):
    tb0 = _wid() * _TPT
    for h in range(2):
        tb = tb0 + h * 32
        pltpu.sync_copy(pos0_hbm.at[pl.ds(tb, 32)], idx_a)
        pltpu.sync_copy(pos1_hbm.at[pl.ds(tb, 32)], idx_b)
        ca = pltpu.async_copy(rows_hbm.at[idx_a], rows_a, sem_a)
        cb = pltpu.async_copy(rows_hbm.at[idx_b], rows_b, sem_b)
        ca.wait()
        cb.wait()
        for j in range(32):
            def add_step(c, _, j=j):
                sl = pl.ds(c * 16, 16)
                out_v[j, sl] = rows_a[j, sl] + rows_b[j, sl]
                return _
            lax.fori_loop(0, _H // 16, add_step, None)
        pltpu.sync_copy(out_v, out_hbm.at[pl.ds(tb, 32)])


def _combine(rows, pos0, pos1):
    return pl.kernel(
        _combine_body,
        out_type=jax.ShapeDtypeStruct((_T, _H), jnp.float32),
        mesh=_mesh(),
        scratch_types=[
            pltpu.VMEM((32,), jnp.int32),
            pltpu.VMEM((32,), jnp.int32),
            pltpu.VMEM((32, _H), jnp.float32),
            pltpu.VMEM((32, _H), jnp.float32),
            pltpu.VMEM((32, _H), jnp.float32),
            pltpu.SemaphoreType.DMA,
            pltpu.SemaphoreType.DMA,
        ],
        compiler_params=_SC_PARAMS,
    )(rows, pos0, pos1)


def _routing_stub(ei, ew):
    cnt = jnp.zeros(16, jnp.int32).at[ei].add(1)
    pcnt = ((cnt + (_B - 1)) >> 8) << 8
    incl = jnp.cumsum(pcnt)
    excl = incl - pcnt
    be = jnp.minimum(jnp.sum(jnp.arange(32)[:, None] * _B >= incl[None, :],
                             axis=1), _E - 1).astype(jnp.int32)
    onehot = ei[:, None] == jnp.arange(_E)[None, :]
    rank = (jnp.cumsum(onehot.astype(jnp.int32), axis=0) - 1)
    rank = jnp.sum(jnp.where(onehot, rank, 0), axis=1)
    pos = excl[ei] + rank
    srctok = jnp.zeros(_NPAD, jnp.int32).at[pos].set(
        (jnp.arange(_P) >> 1).astype(jnp.int32))
    slotw = jnp.zeros(_NPAD, jnp.float32).at[pos].set(ew)
    pos2 = pos.reshape(_T, 2)
    return srctok, slotw, pos2[:, 0], pos2[:, 1], be


def _gemm_stub(xg, w1b, w2b, sw3, be):
    xg3 = xg.reshape(_NB, _B, _H)
    w1g = jnp.take(w1b, be, axis=0)
    w2g = jnp.take(w2b, be, axis=0)
    h1 = jax.nn.gelu(jnp.einsum('ibh,ihf->ibf', xg3, w1g,
                                preferred_element_type=jnp.float32))
    rows = jnp.einsum('ibf,ifh->ibh', h1.astype(jnp.bfloat16), w2g,
                      preferred_element_type=jnp.float32)
    rows = rows * sw3.reshape(_NB, _B, 1)
    return rows.reshape(_NPAD, _H)


def kernel(x, scores, expert_weights, expert_indices, w1, w2):
    del scores
    in_shape = x.shape
    tokens = x.reshape(_T, _H).astype(jnp.bfloat16)
    xi = lax.bitcast_convert_type(tokens.reshape(_T, _HW, 2), jnp.int32)
    ei = expert_indices.reshape(-1).astype(jnp.int32)
    ew = expert_weights.reshape(-1).astype(jnp.float32)
    w1b = w1.astype(jnp.bfloat16)
    w2b = w2.astype(jnp.bfloat16)

    srctok, slotw, pos0, pos1, be32 = _routing(ei, ew)
    xg_i = _gather(xi, srctok)
    xg = lax.bitcast_convert_type(xg_i, jnp.bfloat16).reshape(_NPAD, _H)
    sw3 = slotw.reshape(_NB, 1, _B)
    rows = _gemm(xg, w1b, w2b, sw3, be32[:_NB])
    out = _combine(rows, pos0, pos1)
    return out.reshape(in_shape)


# speedup vs baseline: 1.2665x; 1.2665x over previous
import jax
import jax.numpy as jnp
from jax import lax
from jax.experimental import pallas as pl
from jax.experimental.pallas import tpu as pltpu
from jax.experimental.pallas import tpu_sc as plsc

_T = 2048
_H = 1024
_FF = 4096
_E = 8
_P = 4096
_B = 256
_NPAD = 5888
_NB = 23
_NTILE = 32
_RPT = _NPAD // _NTILE
_TPT = _T // _NTILE

_SC_PARAMS = pltpu.CompilerParams(needs_layout_passes=False)

_mesh = lambda: plsc.VectorSubcoreMesh(
    core_axis_name="c", subcore_axis_name="s", num_cores=2, num_subcores=16)


def _wid():
    return lax.axis_index("s") * 2 + lax.axis_index("c")


def _routing_body(ei_hbm, ew_hbm, srctok_hbm, slotw_hbm, pos0_hbm, pos1_hbm,
                  be_hbm, ei_v, ew_v, srctok_v, slotw_v, pos0_v, pos1_v, be_v,
                  cnt_v, poscnt_v):
    @pl.when(_wid() == 0)
    def _():
        lane = lax.iota(jnp.int32, 16)
        zeros16 = jnp.zeros((16,), jnp.int32)
        pltpu.sync_copy(ei_hbm, ei_v)
        pltpu.sync_copy(ew_hbm, ew_v)
        cnt_v[...] = zeros16

        def hist_step(c, _):
            chunk = ei_v[pl.ds(c * 16, 16)]
            upd = zeros16
            for e in range(_E):
                tot = plsc.all_reduce_population_count(chunk == e)
                upd = upd + jnp.where(lane == e, tot, 0)
            cnt_v[...] = cnt_v[...] + upd
            return _
        lax.fori_loop(0, _P // 16, hist_step, None)

        cnt = cnt_v[...]
        pcnt = ((cnt + (_B - 1)) >> 8) << 8
        incl = plsc.cumsum(pcnt)
        excl = incl - pcnt
        poscnt_v[...] = excl

        for cidx in range(2):
            bevec = zeros16
            for jj in range(16):
                j = cidx * 16 + jj
                cmp = (incl <= j * _B) & (lane < _E)
                cntj = plsc.all_reduce_population_count(cmp)
                bevec = bevec + jnp.where(lane == jj,
                                          jnp.minimum(cntj, _E - 1), 0)
            be_v[pl.ds(cidx * 16, 16)] = bevec

        def zero_step(c, _):
            srctok_v[pl.ds(c * 16, 16)] = zeros16
            slotw_v[pl.ds(c * 16, 16)] = jnp.zeros((16,), jnp.float32)
            return _
        lax.fori_loop(0, _NPAD // 16, zero_step, None)

        def pos_step(c, _):
            chunk = ei_v[pl.ds(c * 16, 16)]
            ewc = ew_v[pl.ds(c * 16, 16)]
            base = plsc.load_gather(poscnt_v, [chunk])
            r = zeros16
            upd = zeros16
            for e in range(_E):
                m = chunk == e
                pref = plsc.cumsum(m.astype(jnp.int32))
                r = r + jnp.where(m, pref - 1, 0)
                tot = plsc.all_reduce_population_count(m)
                upd = upd + jnp.where(lane == e, tot, 0)
            pos = base + r
            poscnt_v[...] = poscnt_v[...] + upd
            pairidx = c * 16 + lane
            tvec = pairidx >> 1
            kpar = pairidx & 1
            plsc.store_scatter(srctok_v, [pos], tvec)
            plsc.store_scatter(slotw_v, [pos], ewc)
            plsc.store_scatter(pos0_v, [tvec], pos, mask=kpar == 0)
            plsc.store_scatter(pos1_v, [tvec], pos, mask=kpar == 1)
            return _
        lax.fori_loop(0, _P // 16, pos_step, None)

        pltpu.sync_copy(srctok_v, srctok_hbm)
        pltpu.sync_copy(slotw_v, slotw_hbm)
        pltpu.sync_copy(pos0_v, pos0_hbm)
        pltpu.sync_copy(pos1_v, pos1_hbm)
        pltpu.sync_copy(be_v, be_hbm)


def _routing(ei, ew):
    return pl.kernel(
        _routing_body,
        out_type=(
            jax.ShapeDtypeStruct((_NPAD,), jnp.int32),
            jax.ShapeDtypeStruct((_NPAD,), jnp.float32),
            jax.ShapeDtypeStruct((_T,), jnp.int32),
            jax.ShapeDtypeStruct((_T,), jnp.int32),
            jax.ShapeDtypeStruct((32,), jnp.int32),
        ),
        mesh=_mesh(),
        scratch_types=[
            pltpu.VMEM((_P,), jnp.int32),
            pltpu.VMEM((_P,), jnp.float32),
            pltpu.VMEM((_NPAD,), jnp.int32),
            pltpu.VMEM((_NPAD,), jnp.float32),
            pltpu.VMEM((_T,), jnp.int32),
            pltpu.VMEM((_T,), jnp.int32),
            pltpu.VMEM((32,), jnp.int32),
            pltpu.VMEM((16,), jnp.int32),
            pltpu.VMEM((16,), jnp.int32),
        ],
        compiler_params=_SC_PARAMS,
    )(ei, ew)


_GCH = (48, 48, 48, 40)


def _gather_body(x_hbm, srctok_hbm, xg_hbm, idx_v, buf0, buf1, sem0, sem1):
    base = _wid() * _RPT
    pltpu.sync_copy(srctok_hbm.at[pl.ds(base, _RPT)], idx_v)
    bufs = (buf0, buf1)
    sems = (sem0, sem1)

    def fire(c):
        sz = _GCH[c]
        return pltpu.async_copy(
            x_hbm.at[idx_v.at[pl.ds(c * 48, sz)]],
            bufs[c % 2].at[pl.ds(0, sz)], sems[c % 2])

    def drain(c, desc):
        sz = _GCH[c]
        desc.wait()
        pltpu.sync_copy(bufs[c % 2].at[pl.ds(0, sz)],
                        xg_hbm.at[pl.ds(base + c * 48, sz)])

    d0 = fire(0)
    d1 = fire(1)
    drain(0, d0)
    d2 = fire(2)
    drain(1, d1)
    d3 = fire(3)
    drain(2, d2)
    drain(3, d3)


def _gather(x2d, srctok):
    return pl.kernel(
        _gather_body,
        out_type=jax.ShapeDtypeStruct((_NPAD, _H), jnp.float32),
        mesh=_mesh(),
        scratch_types=[
            pltpu.VMEM((_RPT,), jnp.int32),
            pltpu.VMEM((48, _H), jnp.float32),
            pltpu.VMEM((48, _H), jnp.float32),
            pltpu.SemaphoreType.DMA,
            pltpu.SemaphoreType.DMA,
        ],
        compiler_params=_SC_PARAMS,
    )(x2d, srctok)


def _gemm_body(be_ref, xg_ref, w1_ref, w2_ref, sw_ref, o_ref, acc_ref):
    k = pl.program_id(0)
    i = pl.program_id(1)
    rows = pl.ds(i * _B, _B)

    @pl.when(k == 0)
    def _init():
        acc_ref[rows, :] = jnp.zeros((_B, _H), jnp.float32)

    xb = xg_ref[...].astype(jnp.bfloat16)
    h1 = jnp.dot(xb, w1_ref[0], preferred_element_type=jnp.float32)
    h1 = jax.nn.gelu(h1)
    part = jnp.dot(h1.astype(jnp.bfloat16), w2_ref[0],
                   preferred_element_type=jnp.float32)
    acc_ref[rows, :] += part

    @pl.when(k == _FF // _H - 1)
    def _flush():
        sw = sw_ref[0, 0, :]
        o_ref[...] = acc_ref[rows, :] * sw[:, None]


def _gemm(xg, w1b, w2b, sw3, be):
    return pl.pallas_call(
        _gemm_body,
        grid_spec=pltpu.PrefetchScalarGridSpec(
            num_scalar_prefetch=1,
            grid=(_FF // _H, _NB),
            in_specs=[
                pl.BlockSpec((_B, _H), lambda k, i, be: (i, 0)),
                pl.BlockSpec((1, _H, _H), lambda k, i, be: (be[i], 0, k)),
                pl.BlockSpec((1, _H, _H), lambda k, i, be: (be[i], k, 0)),
                pl.BlockSpec((1, 1, _B), lambda k, i, be: (i, 0, 0)),
            ],
            out_specs=pl.BlockSpec((_B, _H), lambda k, i, be: (i, 0)),
            scratch_shapes=[pltpu.VMEM((_NPAD, _H), jnp.float32)],
        ),
        out_shape=jax.ShapeDtypeStruct((_NPAD, _H), jnp.float32),
        compiler_params=pltpu.CompilerParams(
            dimension_semantics=("arbitrary", "arbitrary"),
        ),
    )(be, xg, w1b, w2b, sw3)


def _combine_body(rows_hbm, pos0_hbm, pos1_hbm, out_hbm, idx_a, idx_b,
                  rows_a, rows_b, out_v, sem_a, sem_b):
    tb0 = _wid() * _TPT
    for h in range(2):
        tb = tb0 + h * 32
        pltpu.sync_copy(pos0_hbm.at[pl.ds(tb, 32)], idx_a)
        pltpu.sync_copy(pos1_hbm.at[pl.ds(tb, 32)], idx_b)
        ca = pltpu.async_copy(rows_hbm.at[idx_a], rows_a, sem_a)
        cb = pltpu.async_copy(rows_hbm.at[idx_b], rows_b, sem_b)
        ca.wait()
        cb.wait()

        def add_step(c, _):
            sl = pl.ds(c * 16, 16)
            for j in range(32):
                out_v[j, sl] = rows_a[j, sl] + rows_b[j, sl]
            return _
        lax.fori_loop(0, _H // 16, add_step, None)
        pltpu.sync_copy(out_v, out_hbm.at[pl.ds(tb, 32)])


def _combine(rows, pos0, pos1):
    return pl.kernel(
        _combine_body,
        out_type=jax.ShapeDtypeStruct((_T, _H), jnp.float32),
        mesh=_mesh(),
        scratch_types=[
            pltpu.VMEM((32,), jnp.int32),
            pltpu.VMEM((32,), jnp.int32),
            pltpu.VMEM((32, _H), jnp.float32),
            pltpu.VMEM((32, _H), jnp.float32),
            pltpu.VMEM((32, _H), jnp.float32),
            pltpu.SemaphoreType.DMA,
            pltpu.SemaphoreType.DMA,
        ],
        compiler_params=_SC_PARAMS,
    )(rows, pos0, pos1)


def kernel(x, scores, expert_weights, expert_indices, w1, w2):
    del scores
    in_shape = x.shape
    x2d = x.reshape(_T, _H)
    ei = expert_indices.reshape(-1).astype(jnp.int32)
    ew = expert_weights.reshape(-1).astype(jnp.float32)
    w1b = w1.astype(jnp.bfloat16)
    w2b = w2.astype(jnp.bfloat16)

    srctok, slotw, pos0, pos1, be32 = _routing(ei, ew)
    xg = _gather(x2d, srctok)
    sw3 = slotw.reshape(_NB, 1, _B)
    rows = _gemm(xg, w1b, w2b, sw3, be32[:_NB])
    out = _combine(rows, pos0, pos1)
    return out.reshape(in_shape)
